# Initial kernel scaffold; baseline (speedup 1.0000x reference)
import jax
import jax.numpy as jnp
from jax.experimental import pallas as pl


def kernel(x, edge_index, gat_w, att_src, att_dst, gat_bias, W_ih, W_hh, b_ih, b_hh, head_w, head_b):
    raise NotImplementedError("write your pallas kernel here")


# speedup vs baseline: 161.9242x; 161.9242x over previous
import functools

import jax
import jax.numpy as jnp
from jax import lax
from jax.experimental import pallas as pl
from jax.experimental.pallas import tpu as pltpu
from jax.experimental.pallas import tpu_sc as plsc

N = 10000
GAT_H = 8
HID = 64
LANES = 16
EDGE_CHUNK = 8000
NODE_CHUNK = 2000


def _sc_gat_kernel(xr_hbm, src_hbm, dst_hbm, w16_hbm, as16_hbm, ad16_hbm,
                   wpat_hbm, bpat_hbm, lstm_hbm,
                   xv, den, num, sbuf0, sbuf1, dbuf0, dbuf1, aggv, outb, c16,
                   sem_a, sem_b, sem_c, sem_d):
    E = src_hbm.shape[0]
    n_chunks = E // EDGE_CHUNK
    wid = lax.axis_index("s") * 2 + lax.axis_index("c")

    pltpu.sync_copy(xr_hbm.at[wid], xv)
    pltpu.sync_copy(w16_hbm, c16)
    wv = c16[...]
    pltpu.sync_copy(as16_hbm, c16)
    c_src = jnp.sum(c16[...] * wv)
    pltpu.sync_copy(ad16_hbm, c16)
    c_dst = jnp.sum(c16[...] * wv)
    pltpu.sync_copy(wpat_hbm, c16)
    wpat = c16[...]
    pltpu.sync_copy(bpat_hbm, c16)
    bpat = c16[...]

    n_vec = N // LANES

    def max_body(i, mx):
        v = xv[pl.ds(i * LANES, LANES)]
        return jnp.maximum(mx, jnp.abs(v))
    mx = lax.fori_loop(0, n_vec, max_body, jnp.zeros((LANES,), jnp.float32))
    m = (jnp.abs(c_src) + jnp.abs(c_dst)) * jnp.max(mx)

    c_sum = c_src + c_dst
    def self_body(i, _):
        v = xv[pl.ds(i * LANES, LANES)]
        z = c_sum * v
        t = jnp.exp(jnp.maximum(z, 0.2 * z) - m)
        den[pl.ds(i * LANES, LANES)] = t
        num[pl.ds(i * LANES, LANES)] = t * v
        return 0
    lax.fori_loop(0, n_vec, self_body, 0)

    def start(ch):
        slot = ch % 2
        sb, db = (sbuf0, dbuf0) if slot == 0 else (sbuf1, dbuf1)
        sems = (sem_a, sem_b) if slot == 0 else (sem_c, sem_d)
        c1 = pltpu.async_copy(src_hbm.at[pl.ds(ch * EDGE_CHUNK, EDGE_CHUNK)],
                              sb, sems[0])
        c2 = pltpu.async_copy(dst_hbm.at[pl.ds(ch * EDGE_CHUNK, EDGE_CHUNK)],
                              db, sems[1])
        return c1, c2

    UNROLL = 4
    pending = start(0)
    for ch in range(n_chunks):
        slot = ch % 2
        c1, c2 = pending
        c1.wait()
        c2.wait()
        if ch + 1 < n_chunks:
            pending = start(ch + 1)

        sb, db = (sbuf0, dbuf0) if slot == 0 else (sbuf1, dbuf1)

        def edge_body(j, _, sb=sb, db=db):
            base = j * (LANES * UNROLL)
            for u in range(UNROLL):
                off = base + u * LANES
                sv = sb[pl.ds(off, LANES)]
                dv = db[pl.ds(off, LANES)]
                xs = plsc.load_gather(xv, [sv])
                xd = plsc.load_gather(xv, [dv])
                z = c_src * xs + c_dst * xd
                t = jnp.exp(jnp.maximum(z, 0.2 * z) - m)
                plsc.addupdate_scatter(den, [dv], t)
                plsc.addupdate_scatter(num, [dv], t * xs)
            return 0
        lax.fori_loop(0, EDGE_CHUNK // (LANES * UNROLL), edge_body, 0)

    pat = lax.shift_right_logical(lax.iota(jnp.int32, LANES), 3)
    n_out_chunks = N // NODE_CHUNK
    for oc in range(n_out_chunks):
        def agg_body(i, _, oc=oc):
            nb = oc * NODE_CHUNK + i * LANES
            d = den[pl.ds(nb, LANES)]
            n_ = num[pl.ds(nb, LANES)]
            aggv[pl.ds(i * LANES, LANES)] = n_ / (d + 1e-16)
            return 0
        lax.fori_loop(0, NODE_CHUNK // LANES, agg_body, 0)

        def exp_body(k, _):
            idxv = pat + 2 * k
            a = plsc.load_gather(aggv, [idxv])
            r = jnp.maximum(wpat * a + bpat, 0.0)
            outb[pl.ds(k * LANES, LANES)] = r
            return 0
        lax.fori_loop(0, NODE_CHUNK // 2, exp_body, 0)
        pltpu.sync_copy(
            outb, lstm_hbm.at[wid, pl.ds(oc * NODE_CHUNK * GAT_H,
                                         NODE_CHUNK * GAT_H)])


def _sc_gat(xr, src, dst, w16, as16, ad16, wpat, bpat):
    SB = xr.shape[0]
    mesh = plsc.VectorSubcoreMesh(core_axis_name="c", subcore_axis_name="s")
    return pl.kernel(
        _sc_gat_kernel,
        out_type=jax.ShapeDtypeStruct((SB, N * GAT_H), jnp.float32),
        mesh=mesh,
        compiler_params=pltpu.CompilerParams(needs_layout_passes=False),
        scratch_types=[
            pltpu.VMEM((N,), jnp.float32),
            pltpu.VMEM((N,), jnp.float32),
            pltpu.VMEM((N,), jnp.float32),
            pltpu.VMEM((EDGE_CHUNK,), jnp.int32),
            pltpu.VMEM((EDGE_CHUNK,), jnp.int32),
            pltpu.VMEM((EDGE_CHUNK,), jnp.int32),
            pltpu.VMEM((EDGE_CHUNK,), jnp.int32),
            pltpu.VMEM((NODE_CHUNK,), jnp.float32),
            pltpu.VMEM((NODE_CHUNK * GAT_H,), jnp.float32),
            pltpu.VMEM((LANES,), jnp.float32),
            pltpu.SemaphoreType.DMA,
            pltpu.SemaphoreType.DMA,
            pltpu.SemaphoreType.DMA,
            pltpu.SemaphoreType.DMA,
        ],
    )(xr, src, dst, w16, as16, ad16, wpat, bpat)


def _gates_kernel(lstm_ref, wih_ref, out_ref):
    @pl.when(pl.program_id(0) == 0)
    def _():
        out_ref[...] = jnp.zeros_like(out_ref)
    out_ref[...] += lax.dot_general(
        lstm_ref[...], wih_ref[...], (((1,), (1,)), ((), ())),
        preferred_element_type=jnp.float32)


def _gates_matmul(lstm_in, W_ih):
    SB, INP = lstm_in.shape
    G4 = W_ih.shape[0]
    KC = 3200
    grid = (INP // KC,)
    return pl.pallas_call(
        _gates_kernel,
        grid=grid,
        in_specs=[
            pl.BlockSpec((SB, KC), lambda k: (0, k)),
            pl.BlockSpec((G4, KC), lambda k: (0, k)),
        ],
        out_specs=pl.BlockSpec((SB, G4), lambda k: (0, 0)),
        out_shape=jax.ShapeDtypeStruct((SB, G4), jnp.float32),
        compiler_params=pltpu.CompilerParams(
            dimension_semantics=("arbitrary",)),
    )(lstm_in, W_ih)


def _lstm_head_kernel(gates_ref, whh_ref, bih_ref, bhh_ref, headw_ref,
                      headb_ref, out_ref):
    S = gates_ref.shape[0]
    B = gates_ref.shape[1]
    b = bih_ref[...] + bhh_ref[...]

    def step(s, carry):
        h, c = carry
        g = gates_ref[s] + lax.dot_general(
            h, whh_ref[...], (((1,), (1,)), ((), ())),
            preferred_element_type=jnp.float32) + b
        i = jax.nn.sigmoid(g[:, 0:HID])
        f = jax.nn.sigmoid(g[:, HID:2 * HID])
        gg = jnp.tanh(g[:, 2 * HID:3 * HID])
        o = jax.nn.sigmoid(g[:, 3 * HID:4 * HID])
        c = f * c + i * gg
        h = o * jnp.tanh(c)
        return (h, c)

    h0 = jnp.zeros((B, HID), jnp.float32)
    h, _ = lax.fori_loop(0, S, step, (h0, h0))
    out_ref[...] = lax.dot_general(
        h, headw_ref[...], (((1,), (1,)), ((), ())),
        preferred_element_type=jnp.float32) + headb_ref[...]


def _lstm_head(gates, W_hh, b_ih, b_hh, head_w, head_b):
    S, B, G4 = gates.shape
    return pl.pallas_call(
        _lstm_head_kernel,
        out_shape=jax.ShapeDtypeStruct((B, N), jnp.float32),
    )(gates, W_hh, b_ih.reshape(1, G4), b_hh.reshape(1, G4),
      head_w, head_b.reshape(1, N))


def kernel(x, edge_index, gat_w, att_src, att_dst, gat_bias, W_ih, W_hh,
           b_ih, b_hh, head_w, head_b):
    B, S, Nx = x.shape
    xr = x.transpose(1, 0, 2).reshape(S * B, Nx)
    src = edge_index[0]
    dst = edge_index[1]
    w = gat_w[:, 0]
    pad8 = jnp.zeros((GAT_H,), jnp.float32)
    w16 = jnp.concatenate([w, pad8])
    as16 = jnp.concatenate([att_src, pad8])
    ad16 = jnp.concatenate([att_dst, pad8])
    wpat = jnp.concatenate([w, w])
    bpat = jnp.concatenate([gat_bias, gat_bias])

    lstm_in = _sc_gat(xr, src, dst, w16, as16, ad16, wpat, bpat)
    gates = _gates_matmul(lstm_in, W_ih)
    gates = gates.reshape(S, B, 4 * HID)
    return _lstm_head(gates, W_hh, b_ih, b_hh, head_w, head_b)
